# Initial kernel scaffold; baseline (speedup 1.0000x reference)
import jax
import jax.numpy as jnp
from jax.experimental import pallas as pl


def kernel(token_ids, embeddings):
    raise NotImplementedError("write your pallas kernel here")


# speedup vs baseline: 8.2129x; 8.2129x over previous
import functools

import jax
import jax.numpy as jnp
from jax import lax
from jax.experimental import pallas as pl
from jax.experimental.pallas import tpu as pltpu
from jax.experimental.pallas import tpu_sc as plsc

NUM_TOK = 4096 * 200
DIM = 128

_info = plsc.get_sparse_core_info()
_NC = _info.num_cores
_NS = _info.num_subcores
_NW = _NC * _NS

ROWS_PER_W = NUM_TOK // _NW
IDX_L = 128
STREAMS = 4
IDX_ROWS = 8
CHUNK = IDX_L * IDX_ROWS
HALF = IDX_L * STREAMS
N_CHUNKS = ROWS_PER_W // CHUNK

_mesh = plsc.VectorSubcoreMesh(core_axis_name="c", subcore_axis_name="s")


@functools.partial(
    pl.kernel,
    mesh=_mesh,
    out_type=jax.ShapeDtypeStruct((NUM_TOK, DIM), jnp.float32),
    scratch_types=[
        pltpu.VMEM((IDX_ROWS, IDX_L), jnp.int32),
        pltpu.VMEM((HALF, DIM), jnp.float32),
        pltpu.SemaphoreType.DMA,
    ],
)
def _emb_lookup(ids_hbm, table_hbm, out_hbm, idx_v, rows_v, sem):
    wid = lax.axis_index("s") * _NC + lax.axis_index("c")
    row0 = wid * ROWS_PER_W
    idx_row0 = row0 // IDX_L

    def chunk_body(i, _):
        idx_off = pl.multiple_of(idx_row0 + i * IDX_ROWS, IDX_ROWS)
        pltpu.sync_copy(ids_hbm.at[pl.ds(idx_off, IDX_ROWS)], idx_v)
        for h in range(IDX_ROWS // STREAMS):
            copies = []
            for j in range(STREAMS):
                copies.append(pltpu.async_copy(
                    table_hbm.at[idx_v.at[h * STREAMS + j]],
                    rows_v.at[pl.ds(j * IDX_L, IDX_L)],
                    sem))
            for c in copies:
                c.wait()
            pltpu.sync_copy(
                rows_v, out_hbm.at[pl.ds(row0 + i * CHUNK + h * HALF, HALF)])
        return 0

    lax.fori_loop(0, N_CHUNKS, chunk_body, 0)


def kernel(token_ids, embeddings):
    flat_ids = token_ids.reshape(NUM_TOK // IDX_L, IDX_L).astype(jnp.int32)
    out = _emb_lookup(flat_ids, embeddings)
    return out.reshape(token_ids.shape[0], token_ids.shape[1], DIM)


# speedup vs baseline: 8.9072x; 1.0845x over previous
import functools

import jax
import jax.numpy as jnp
from jax import lax
from jax.experimental import pallas as pl
from jax.experimental.pallas import tpu as pltpu
from jax.experimental.pallas import tpu_sc as plsc

NUM_TOK = 4096 * 200
DIM = 128

_info = plsc.get_sparse_core_info()
_NC = _info.num_cores
_NS = _info.num_subcores
_NW = _NC * _NS

ROWS_PER_W = NUM_TOK // _NW
IDX_L = 128
STREAMS = 2
IDX_ROWS = 8
CHUNK = IDX_L * IDX_ROWS
HALF = IDX_L * STREAMS
NBUF = 2
N_CHUNKS = ROWS_PER_W // CHUNK

_mesh = plsc.VectorSubcoreMesh(core_axis_name="c", subcore_axis_name="s")


@functools.partial(
    pl.kernel,
    mesh=_mesh,
    out_type=jax.ShapeDtypeStruct((NUM_TOK, DIM), jnp.float32),
    scratch_types=[
        pltpu.VMEM((IDX_ROWS, IDX_L), jnp.int32),
        pltpu.VMEM((NBUF, HALF, DIM), jnp.float32),
        pltpu.SemaphoreType.DMA,
        pltpu.SemaphoreType.DMA,
    ],
)
def _emb_lookup(ids_hbm, table_hbm, out_hbm, idx_v, rows_v, g_sem, o_sem):
    wid = lax.axis_index("s") * _NC + lax.axis_index("c")
    row0 = wid * ROWS_PER_W
    idx_row0 = row0 // IDX_L
    n_halves = IDX_ROWS // STREAMS

    def out_wait(buf):
        pltpu.make_async_copy(
            rows_v.at[buf], out_hbm.at[pl.ds(0, HALF)], o_sem).wait()

    def chunk_body(i, _):
        idx_off = pl.multiple_of(idx_row0 + i * IDX_ROWS, IDX_ROWS)
        pltpu.sync_copy(ids_hbm.at[pl.ds(idx_off, IDX_ROWS)], idx_v)
        for h in range(n_halves):
            buf = h % NBUF
            if h >= NBUF:
                out_wait(buf)
            else:
                @pl.when(i > 0)
                def _():
                    out_wait(buf)
            copies = []
            for j in range(STREAMS):
                copies.append(pltpu.async_copy(
                    table_hbm.at[idx_v.at[h * STREAMS + j]],
                    rows_v.at[(buf, pl.ds(j * IDX_L, IDX_L))],
                    g_sem))
            for c in copies:
                c.wait()
            pltpu.async_copy(
                rows_v.at[buf],
                out_hbm.at[pl.ds(row0 + i * CHUNK + h * HALF, HALF)],
                o_sem)
        return 0

    lax.fori_loop(0, N_CHUNKS, chunk_body, 0)
    for buf in range(NBUF):
        out_wait(buf)


def kernel(token_ids, embeddings):
    flat_ids = token_ids.reshape(NUM_TOK // IDX_L, IDX_L).astype(jnp.int32)
    out = _emb_lookup(flat_ids, embeddings)
    return out.reshape(token_ids.shape[0], token_ids.shape[1], DIM)


# speedup vs baseline: 9.1397x; 1.0261x over previous
import functools

import jax
import jax.numpy as jnp
from jax import lax
from jax.experimental import pallas as pl
from jax.experimental.pallas import tpu as pltpu
from jax.experimental.pallas import tpu_sc as plsc

NUM_TOK = 4096 * 200
DIM = 128

_info = plsc.get_sparse_core_info()
_NC = _info.num_cores
_NS = _info.num_subcores
_NW = _NC * _NS

ROWS_PER_W = NUM_TOK // _NW
IDX_L = 128
STREAMS = 2
IDX_ROWS = 8
CHUNK = IDX_L * IDX_ROWS
HALF = IDX_L * STREAMS
NBUF = 2
N_CHUNKS = ROWS_PER_W // CHUNK

_mesh = plsc.VectorSubcoreMesh(core_axis_name="c", subcore_axis_name="s")


@functools.partial(
    pl.kernel,
    mesh=_mesh,
    out_type=jax.ShapeDtypeStruct((NUM_TOK, DIM), jnp.float32),
    scratch_types=[
        pltpu.VMEM((2, IDX_ROWS, IDX_L), jnp.int32),
        pltpu.VMEM((NBUF, HALF, DIM), jnp.float32),
        pltpu.SemaphoreType.DMA,
        pltpu.SemaphoreType.DMA,
        pltpu.SemaphoreType.DMA,
    ],
)
def _emb_lookup(ids_hbm, table_hbm, out_hbm, idx_v, rows_v, g_sem, o_sem,
                i_sem):
    wid = lax.axis_index("s") * _NC + lax.axis_index("c")
    row0 = wid * ROWS_PER_W
    idx_row0 = row0 // IDX_L
    n_halves = IDX_ROWS // STREAMS

    def out_wait():
        pltpu.make_async_copy(
            rows_v.at[0], out_hbm.at[pl.ds(0, HALF)], o_sem).wait()

    def gather_wait():
        pltpu.make_async_copy(
            table_hbm.at[idx_v.at[(0, 0)]],
            rows_v.at[(0, pl.ds(0, IDX_L))], g_sem).wait()

    def fire_gathers(islot, h, buf):
        for j in range(STREAMS):
            pltpu.async_copy(
                table_hbm.at[idx_v.at[(islot, h * STREAMS + j)]],
                rows_v.at[(buf, pl.ds(j * IDX_L, IDX_L))],
                g_sem)

    pltpu.sync_copy(
        ids_hbm.at[pl.ds(pl.multiple_of(idx_row0, IDX_ROWS), IDX_ROWS)],
        idx_v.at[0])
    fire_gathers(0, 0, 0)

    def chunk_body(i, _):
        islot = i % 2
        nslot = (i + 1) % 2
        @pl.when(i < N_CHUNKS - 1)
        def _():
            idx_off = pl.multiple_of(idx_row0 + (i + 1) * IDX_ROWS, IDX_ROWS)
            pltpu.async_copy(ids_hbm.at[pl.ds(idx_off, IDX_ROWS)],
                             idx_v.at[nslot], i_sem)

        for h in range(n_halves):
            buf = h % NBUF
            nb = (h + 1) % NBUF
            if h < n_halves - 1:
                if h == 0:
                    @pl.when(i > 0)
                    def _():
                        out_wait()
                else:
                    out_wait()
                fire_gathers(islot, h + 1, nb)
            else:
                out_wait()
                @pl.when(i < N_CHUNKS - 1)
                def _():
                    pltpu.make_async_copy(
                        ids_hbm.at[pl.ds(0, IDX_ROWS)], idx_v.at[nslot],
                        i_sem).wait()
                    fire_gathers(nslot, 0, nb)
            for _j in range(STREAMS):
                gather_wait()
            pltpu.async_copy(
                rows_v.at[buf],
                out_hbm.at[pl.ds(row0 + i * CHUNK + h * HALF, HALF)],
                o_sem)
        return 0

    lax.fori_loop(0, N_CHUNKS, chunk_body, 0)
    out_wait()


def kernel(token_ids, embeddings):
    flat_ids = token_ids.reshape(NUM_TOK // IDX_L, IDX_L).astype(jnp.int32)
    out = _emb_lookup(flat_ids, embeddings)
    return out.reshape(token_ids.shape[0], token_ids.shape[1], DIM)


# speedup vs baseline: 9.2080x; 1.0075x over previous
import functools

import jax
import jax.numpy as jnp
from jax import lax
from jax.experimental import pallas as pl
from jax.experimental.pallas import tpu as pltpu
from jax.experimental.pallas import tpu_sc as plsc

NUM_TOK = 4096 * 200
DIM = 128

_info = plsc.get_sparse_core_info()
_NC = _info.num_cores
_NS = _info.num_subcores
_NW = _NC * _NS

ROWS_PER_W = NUM_TOK // _NW
IDX_L = 128
IDX_ROWS = 8
CHUNK = IDX_L * IDX_ROWS
HALF = IDX_L
NBUF = 4
N_STAGES = IDX_ROWS
N_CHUNKS = ROWS_PER_W // CHUNK

_mesh = plsc.VectorSubcoreMesh(core_axis_name="c", subcore_axis_name="s")


@functools.partial(
    pl.kernel,
    mesh=_mesh,
    out_type=jax.ShapeDtypeStruct((NUM_TOK, DIM), jnp.float32),
    scratch_types=[
        pltpu.VMEM((2, IDX_ROWS, IDX_L), jnp.int32),
        pltpu.VMEM((NBUF, HALF, DIM), jnp.float32),
        pltpu.SemaphoreType.DMA,
        pltpu.SemaphoreType.DMA,
        pltpu.SemaphoreType.DMA,
    ],
)
def _emb_lookup(ids_hbm, table_hbm, out_hbm, idx_v, rows_v, g_sem, o_sem,
                i_sem):
    wid = lax.axis_index("s") * _NC + lax.axis_index("c")
    row0 = wid * ROWS_PER_W
    idx_row0 = row0 // IDX_L

    def out_wait():
        pltpu.make_async_copy(
            rows_v.at[0], out_hbm.at[pl.ds(0, HALF)], o_sem).wait()

    def gather_wait():
        pltpu.make_async_copy(
            table_hbm.at[idx_v.at[(0, 0)]],
            rows_v.at[0], g_sem).wait()

    def fire_gather(islot, h, buf):
        pltpu.async_copy(table_hbm.at[idx_v.at[(islot, h)]],
                         rows_v.at[buf], g_sem)

    pltpu.sync_copy(
        ids_hbm.at[pl.ds(pl.multiple_of(idx_row0, IDX_ROWS), IDX_ROWS)],
        idx_v.at[0])
    fire_gather(0, 0, 0)
    fire_gather(0, 1, 1)

    def chunk_body(i, _):
        islot = i % 2
        nslot = (i + 1) % 2
        @pl.when(i < N_CHUNKS - 1)
        def _():
            idx_off = pl.multiple_of(idx_row0 + (i + 1) * IDX_ROWS, IDX_ROWS)
            pltpu.async_copy(ids_hbm.at[pl.ds(idx_off, IDX_ROWS)],
                             idx_v.at[nslot], i_sem)

        for h in range(N_STAGES):
            buf = h % NBUF
            tbuf = (h + 2) % NBUF
            if h >= 2:
                out_wait()
            else:
                @pl.when(i > 0)
                def _():
                    out_wait()
            if h < N_STAGES - 2:
                fire_gather(islot, h + 2, tbuf)
            elif h == N_STAGES - 2:
                @pl.when(i < N_CHUNKS - 1)
                def _():
                    pltpu.make_async_copy(
                        ids_hbm.at[pl.ds(0, IDX_ROWS)], idx_v.at[nslot],
                        i_sem).wait()
                    fire_gather(nslot, 0, tbuf)
            else:
                @pl.when(i < N_CHUNKS - 1)
                def _():
                    fire_gather(nslot, 1, tbuf)
            gather_wait()
            pltpu.async_copy(
                rows_v.at[buf],
                out_hbm.at[pl.ds(row0 + i * CHUNK + h * HALF, HALF)],
                o_sem)
        return 0

    lax.fori_loop(0, N_CHUNKS, chunk_body, 0)
    out_wait()
    out_wait()


def kernel(token_ids, embeddings):
    flat_ids = token_ids.reshape(NUM_TOK // IDX_L, IDX_L).astype(jnp.int32)
    out = _emb_lookup(flat_ids, embeddings)
    return out.reshape(token_ids.shape[0], token_ids.shape[1], DIM)


# speedup vs baseline: 9.5975x; 1.0423x over previous
import functools

import jax
import jax.numpy as jnp
from jax import lax
from jax.experimental import pallas as pl
from jax.experimental.pallas import tpu as pltpu
from jax.experimental.pallas import tpu_sc as plsc

NUM_TOK = 4096 * 200
DIM = 128

_info = plsc.get_sparse_core_info()
_NC = _info.num_cores
_NS = _info.num_subcores
_NW = _NC * _NS

ROWS_PER_W = NUM_TOK // _NW
IDX_L = 128
IDX_ROWS = 8
CHUNK = IDX_L * IDX_ROWS
HALF = IDX_L
N_STAGES = IDX_ROWS
N_CHUNKS = ROWS_PER_W // CHUNK

_mesh = plsc.VectorSubcoreMesh(core_axis_name="c", subcore_axis_name="s")


@functools.partial(
    pl.kernel,
    mesh=_mesh,
    out_type=jax.ShapeDtypeStruct((NUM_TOK, DIM), jnp.float32),
    scratch_types=[
        pltpu.VMEM((2, IDX_ROWS, IDX_L), jnp.int32),
        pltpu.VMEM((4, HALF, DIM), jnp.float32),
        pltpu.VMEM((2, HALF, DIM), jnp.float32),
        pltpu.VMEM_SHARED((_NS, HALF, DIM), jnp.float32),
        pltpu.SemaphoreType.DMA,
        pltpu.SemaphoreType.DMA,
        pltpu.SemaphoreType.DMA,
        pltpu.SemaphoreType.DMA,
        pltpu.SemaphoreType.DMA,
        pltpu.SemaphoreType.DMA,
    ],
)
def _emb_lookup(ids_hbm, table_hbm, out_hbm, idx_v, rows_a, rows_b, shared_v,
                ga_sem, gb_sem, oa_sem, c_sem, d_sem, i_sem):
    wid = lax.axis_index("s") * _NC + lax.axis_index("c")
    sid = lax.axis_index("s")
    row0 = wid * ROWS_PER_W
    idx_row0 = row0 // IDX_L

    def wait_oa():
        pltpu.make_async_copy(
            rows_a.at[0], out_hbm.at[pl.ds(0, HALF)], oa_sem).wait()

    def wait_c():
        pltpu.make_async_copy(
            rows_b.at[0], shared_v.at[sid], c_sem).wait()

    def wait_d():
        pltpu.make_async_copy(
            shared_v.at[sid], out_hbm.at[pl.ds(0, HALF)], d_sem).wait()

    def gather_wait(sem):
        pltpu.make_async_copy(
            table_hbm.at[idx_v.at[(0, 0)]], rows_a.at[0], sem).wait()

    def fire_gather(islot, h, dst, sem):
        pltpu.async_copy(table_hbm.at[idx_v.at[(islot, h)]], dst, sem)

    pltpu.sync_copy(
        ids_hbm.at[pl.ds(pl.multiple_of(idx_row0, IDX_ROWS), IDX_ROWS)],
        idx_v.at[0])
    fire_gather(0, 0, rows_a.at[0], ga_sem)
    fire_gather(0, 1, rows_b.at[0], gb_sem)

    def chunk_body(i, _):
        islot = i % 2
        nslot = (i + 1) % 2
        @pl.when(i < N_CHUNKS - 1)
        def _():
            idx_off = pl.multiple_of(idx_row0 + (i + 1) * IDX_ROWS, IDX_ROWS)
            pltpu.async_copy(ids_hbm.at[pl.ds(idx_off, IDX_ROWS)],
                             idx_v.at[nslot], i_sem)

        for h in range(N_STAGES):
            k = h >> 1
            off = row0 + i * CHUNK + h * HALF
            if h % 2 == 0:
                if k == 3:
                    wait_oa()
                else:
                    @pl.when(i > 0)
                    def _():
                        wait_oa()
                nxt = rows_a.at[(k + 1) % 4]
                if h < N_STAGES - 2:
                    fire_gather(islot, h + 2, nxt, ga_sem)
                else:
                    @pl.when(i < N_CHUNKS - 1)
                    def _():
                        pltpu.make_async_copy(
                            ids_hbm.at[pl.ds(0, IDX_ROWS)], idx_v.at[nslot],
                            i_sem).wait()
                        fire_gather(nslot, 0, rows_a.at[0], ga_sem)
                gather_wait(ga_sem)
                pltpu.async_copy(rows_a.at[k], out_hbm.at[pl.ds(off, HALF)],
                                 oa_sem)
            else:
                nxt = rows_b.at[(k + 1) % 2]
                if h < N_STAGES - 2:
                    fire_gather(islot, h + 2, nxt, gb_sem)
                else:
                    @pl.when(i < N_CHUNKS - 1)
                    def _():
                        fire_gather(nslot, 1, rows_b.at[0], gb_sem)
                gather_wait(gb_sem)
                if k >= 1:
                    wait_d()
                else:
                    @pl.when(i > 0)
                    def _():
                        wait_d()
                pltpu.async_copy(rows_b.at[k % 2], shared_v.at[sid], c_sem)
                wait_c()
                pltpu.async_copy(shared_v.at[sid],
                                 out_hbm.at[pl.ds(off, HALF)], d_sem)
        return 0

    lax.fori_loop(0, N_CHUNKS, chunk_body, 0)
    wait_oa()
    wait_oa()
    wait_oa()
    wait_d()


def kernel(token_ids, embeddings):
    flat_ids = token_ids.reshape(NUM_TOK // IDX_L, IDX_L).astype(jnp.int32)
    out = _emb_lookup(flat_ids, embeddings)
    return out.reshape(token_ids.shape[0], token_ids.shape[1], DIM)
